# baseline (speedup 1.0000x reference)
import functools

import jax
import jax.numpy as jnp
from jax import lax
from jax.experimental import pallas as pl
from jax.experimental.pallas import tpu as pltpu
from jax.experimental.pallas import tpu_sc as plsc

N = 10000
E = 320000
HID = 128
NPAD = 10240
EPAD = 327680
CW = 16
NC = 2
NS = 16
NW = NC * NS
EPT = EPAD // NW
CH = 64
NCHUNK = EPT // CH
RPT = NPAD // NS

_f32 = jnp.float32


def _silu(x):
    return x / (1.0 + jnp.exp(-x))


def _prep_body(h_ref, t_ref, WaT, WbT, WtT, be1, A_ref, B_ref):
    h = h_ref[...]
    A_ref[...] = (jnp.dot(h, WaT[...], preferred_element_type=_f32)
                  + jnp.dot(t_ref[...], WtT[...], preferred_element_type=_f32)
                  + be1[...])
    B_ref[...] = jnp.dot(h, WbT[...], preferred_element_type=_f32)


def _prep_call(hp, tp, WaT, WbT, WtT, be1):
    blk = 2048
    grid = NPAD // blk
    w_spec = pl.BlockSpec((HID, HID), lambda i: (0, 0))
    b_spec = pl.BlockSpec((1, HID), lambda i: (0, 0))
    r_spec = pl.BlockSpec((blk, HID), lambda i: (i, 0))
    return pl.pallas_call(
        _prep_body,
        grid=(grid,),
        in_specs=[r_spec, r_spec, w_spec, w_spec, w_spec, b_spec],
        out_specs=[r_spec, r_spec],
        out_shape=[jax.ShapeDtypeStruct((NPAD, HID), _f32),
                   jax.ShapeDtypeStruct((NPAD, HID), _f32)],
    )(hp, tp, WaT, WbT, WtT, be1)


def _gather_sc(A, B, px, py, pz, row3, col3,
               GA, GB, dxo, dyo, dzo,
               pxt, pyt, pzt, rowf, colf, bufA, bufB, dxf, dyf, dzf,
               semA, semB):
    cid = lax.axis_index("c")
    sid = lax.axis_index("s")
    wid = cid * NS + sid
    pltpu.sync_copy(px, pxt)
    pltpu.sync_copy(py, pyt)
    pltpu.sync_copy(pz, pzt)
    pltpu.sync_copy(row3.at[wid], rowf)
    pltpu.sync_copy(col3.at[wid], colf)

    def chunk(k, carry):
        base = pl.multiple_of(wid * EPT, CH) + k * CH
        cpA = pltpu.async_copy(A.at[rowf.at[k]], bufA, semA)
        cpB = pltpu.async_copy(B.at[colf.at[k]], bufB, semB)
        for i in range(CH // 16):
            r = rowf[k, pl.ds(i * 16, 16)]
            c = colf[k, pl.ds(i * 16, 16)]
            o = pl.ds(k * CH + i * 16, 16)
            dxf[o] = plsc.load_gather(pxt, [r]) - plsc.load_gather(pxt, [c])
            dyf[o] = plsc.load_gather(pyt, [r]) - plsc.load_gather(pyt, [c])
            dzf[o] = plsc.load_gather(pzt, [r]) - plsc.load_gather(pzt, [c])
        cpA.wait()
        cpB.wait()
        pltpu.sync_copy(bufA, GA.at[pl.ds(base, CH)])
        pltpu.sync_copy(bufB, GB.at[pl.ds(base, CH)])
        return carry

    lax.fori_loop(0, NCHUNK, chunk, 0)
    tbase = pl.multiple_of(wid * EPT, CH)
    pltpu.sync_copy(dxf, dxo.at[pl.ds(tbase, EPT)])
    pltpu.sync_copy(dyf, dyo.at[pl.ds(tbase, EPT)])
    pltpu.sync_copy(dzf, dzo.at[pl.ds(tbase, EPT)])


def _gather_call(A, B, px, py, pz, row3, col3):
    mesh = plsc.VectorSubcoreMesh(core_axis_name="c", subcore_axis_name="s",
                                  num_cores=NC, num_subcores=NS)
    f = functools.partial(
        pl.kernel,
        out_type=[jax.ShapeDtypeStruct((EPAD, HID), _f32),
                  jax.ShapeDtypeStruct((EPAD, HID), _f32),
                  jax.ShapeDtypeStruct((EPAD,), _f32),
                  jax.ShapeDtypeStruct((EPAD,), _f32),
                  jax.ShapeDtypeStruct((EPAD,), _f32)],
        mesh=mesh,
        scratch_types=[
            pltpu.VMEM((NPAD,), _f32),
            pltpu.VMEM((NPAD,), _f32),
            pltpu.VMEM((NPAD,), _f32),
            pltpu.VMEM((NCHUNK, CH), jnp.int32),
            pltpu.VMEM((NCHUNK, CH), jnp.int32),
            pltpu.VMEM((CH, HID), _f32),
            pltpu.VMEM((CH, HID), _f32),
            pltpu.VMEM((EPT,), _f32),
            pltpu.VMEM((EPT,), _f32),
            pltpu.VMEM((EPT,), _f32),
            pltpu.SemaphoreType.DMA,
            pltpu.SemaphoreType.DMA,
        ],
        compiler_params=pltpu.CompilerParams(needs_layout_passes=False),
    )(_gather_sc)
    return f(A, B, px, py, pz, row3, col3)


def _edge_body(GA, GB, dx, dy, dz, We2T, be2, Wc1T, bc1, Wc2T, wd,
               mij_o, cwx_o, cwy_o, cwz_o):
    dxv = dx[...]
    dyv = dy[...]
    dzv = dz[...]
    d2 = dxv * dxv + dyv * dyv + dzv * dzv + 1e-8
    dist = jnp.sqrt(d2)
    pre = GA[...] + GB[...] + dist * wd[...]
    m = _silu(pre)
    mij = _silu(jnp.dot(m, We2T[...], preferred_element_type=_f32) + be2[...])
    cwv = _silu(jnp.dot(mij, Wc1T[...], preferred_element_type=_f32)
                + bc1[...])
    cw = jnp.dot(cwv, Wc2T[...], preferred_element_type=_f32)
    s = cw / (dist + 1e-8)
    mij_o[...] = mij
    cwx_o[...] = s * dxv
    cwy_o[...] = s * dyv
    cwz_o[...] = s * dzv


def _edge_call(GA, GB, dx, dy, dz, We2T, be2, Wc1T, bc1, Wc2T, wd):
    blk = 1024
    grid = EPAD // blk
    g_spec = pl.BlockSpec((blk, HID), lambda i: (i, 0))
    c_spec = pl.BlockSpec((blk, 1), lambda i: (i, 0))
    w_spec = pl.BlockSpec((HID, HID), lambda i: (0, 0))
    b_spec = pl.BlockSpec((1, HID), lambda i: (0, 0))
    w2_spec = pl.BlockSpec((HID, 1), lambda i: (0, 0))
    return pl.pallas_call(
        _edge_body,
        grid=(grid,),
        in_specs=[g_spec, g_spec, c_spec, c_spec, c_spec, w_spec, b_spec,
                  w_spec, b_spec, w2_spec, b_spec],
        out_specs=[g_spec, c_spec, c_spec, c_spec],
        out_shape=[jax.ShapeDtypeStruct((EPAD, HID), _f32),
                   jax.ShapeDtypeStruct((EPAD, 1), _f32),
                   jax.ShapeDtypeStruct((EPAD, 1), _f32),
                   jax.ShapeDtypeStruct((EPAD, 1), _f32)],
    )(GA, GB, dx, dy, dz, We2T, be2, Wc1T, bc1, Wc2T, wd)


def _msg_scatter_sc(row3, mij, z2d, mpart, macc, rowf, mbuf):
    cid = lax.axis_index("c")
    sid = lax.axis_index("s")
    wid = cid * NS + sid
    pltpu.sync_copy(z2d.at[pl.ds(sid * RPT, RPT)],
                    macc.at[pl.ds(sid * RPT, RPT)])
    pltpu.sync_copy(row3.at[wid], rowf)
    plsc.subcore_barrier()

    def chunk(k, carry):
        base = pl.multiple_of(wid * EPT, CH) + k * CH
        pltpu.sync_copy(mij.at[pl.ds(base, CH)], mbuf)
        pltpu.sync_copy(mbuf, macc.at[rowf.at[k]], add=True)
        return carry

    lax.fori_loop(0, NCHUNK, chunk, 0)
    plsc.subcore_barrier()
    pltpu.sync_copy(macc.at[pl.ds(sid * RPT, RPT)],
                    mpart.at[cid, pl.ds(sid * RPT, RPT)])


def _msg_scatter_call(row3, mij, z2d):
    mesh = plsc.VectorSubcoreMesh(core_axis_name="c", subcore_axis_name="s",
                                  num_cores=NC, num_subcores=NS)
    f = functools.partial(
        pl.kernel,
        out_type=[jax.ShapeDtypeStruct((NC, NPAD, HID), _f32)],
        mesh=mesh,
        scratch_types=[
            pltpu.MemorySpace.VMEM_SHARED((NPAD, HID), _f32),
            pltpu.VMEM((NCHUNK, CH), jnp.int32),
            pltpu.VMEM((CH, HID), _f32),
        ],
        compiler_params=pltpu.CompilerParams(needs_layout_passes=False),
    )(_msg_scatter_sc)
    [mpart] = f(row3, mij, z2d)
    return mpart


def _coord_scatter_sc(row3, cwx, cwy, cwz, cpart,
                      acc3, rowf, cxb, cyb, czb):
    cid = lax.axis_index("c")
    sid = lax.axis_index("s")
    wid = cid * NS + sid

    def zinit(i, c2):
        o = pl.ds(i * 16, 16)
        z = jnp.zeros((16,), _f32)
        acc3[0, o] = z
        acc3[1, o] = z
        acc3[2, o] = z
        return c2

    lax.fori_loop(0, NPAD // 16, zinit, 0)
    tbase = pl.multiple_of(wid * EPT, CH)
    pltpu.sync_copy(row3.at[wid], rowf)
    pltpu.sync_copy(cwx.at[pl.ds(tbase, EPT)], cxb)
    pltpu.sync_copy(cwy.at[pl.ds(tbase, EPT)], cyb)
    pltpu.sync_copy(cwz.at[pl.ds(tbase, EPT)], czb)
    d0 = jnp.zeros((16,), jnp.int32)
    d1 = jnp.ones((16,), jnp.int32)
    d2 = jnp.full((16,), 2, jnp.int32)

    def chunk(k, carry):
        def vec(i, c2):
            o = pl.ds(k * CH + i * 16, 16)
            r = rowf[k, pl.ds(i * 16, 16)]
            plsc.addupdate_scatter(acc3, [d0, r], cxb[o])
            plsc.addupdate_scatter(acc3, [d1, r], cyb[o])
            plsc.addupdate_scatter(acc3, [d2, r], czb[o])
            return c2

        return lax.fori_loop(0, CH // 16, vec, carry)

    lax.fori_loop(0, NCHUNK, chunk, 0)
    pltpu.sync_copy(acc3, cpart.at[wid, pl.ds(0, 3)])


def _coord_scatter_call(row3, cwx, cwy, cwz):
    mesh = plsc.VectorSubcoreMesh(core_axis_name="c", subcore_axis_name="s",
                                  num_cores=NC, num_subcores=NS)
    f = functools.partial(
        pl.kernel,
        out_type=[jax.ShapeDtypeStruct((NW, 8, NPAD), _f32)],
        mesh=mesh,
        scratch_types=[
            pltpu.VMEM((3, NPAD), _f32),
            pltpu.VMEM((NCHUNK, CH), jnp.int32),
            pltpu.VMEM((EPT,), _f32),
            pltpu.VMEM((EPT,), _f32),
            pltpu.VMEM((EPT,), _f32),
        ],
        compiler_params=pltpu.CompilerParams(needs_layout_passes=False),
    )(_coord_scatter_sc)
    [cpart] = f(row3, cwx, cwy, cwz)
    return cpart


def _node_body(h_ref, t_ref, p0, p1, WhT, WmT, Wt2T, bn1, Wn2T, bn2, gam, bet,
               out):
    msg = p0[0] + p1[0]
    hv = h_ref[...]
    hm = _silu(jnp.dot(hv, WhT[...], preferred_element_type=_f32)
               + jnp.dot(msg, WmT[...], preferred_element_type=_f32)
               + jnp.dot(t_ref[...], Wt2T[...], preferred_element_type=_f32)
               + bn1[...])
    y = hv + jnp.dot(hm, Wn2T[...], preferred_element_type=_f32) + bn2[...]
    mu = jnp.mean(y, axis=1, keepdims=True)
    var = jnp.mean((y - mu) ** 2, axis=1, keepdims=True)
    out[...] = (y - mu) * lax.rsqrt(var + 1e-5) * gam[...] + bet[...]


def _node_call(h, t_emb, mpart, WhT, WmT, Wt2T, bn1, Wn2T, bn2, gam, bet):
    blk = 2000
    grid = N // blk
    r_spec = pl.BlockSpec((blk, HID), lambda i: (i, 0))
    p0_spec = pl.BlockSpec((1, blk, HID), lambda i: (0, i, 0))
    p1_spec = pl.BlockSpec((1, blk, HID), lambda i: (1, i, 0))
    w_spec = pl.BlockSpec((HID, HID), lambda i: (0, 0))
    b_spec = pl.BlockSpec((1, HID), lambda i: (0, 0))
    return pl.pallas_call(
        _node_body,
        grid=(grid,),
        in_specs=[r_spec, r_spec, p0_spec, p1_spec, w_spec, w_spec, w_spec,
                  b_spec, w_spec, b_spec, b_spec, b_spec],
        out_specs=r_spec,
        out_shape=jax.ShapeDtypeStruct((N, HID), _f32),
    )(h, t_emb, mpart, mpart, WhT, WmT, Wt2T, bn1, Wn2T, bn2, gam, bet)


def _pos_body(pos8, cp, out):
    out[...] = pos8[...] + jnp.sum(cp[...], axis=0)


def _pos_call(pos8, cpart):
    return pl.pallas_call(
        _pos_body,
        in_specs=[pl.BlockSpec((8, NPAD), lambda: (0, 0)),
                  pl.BlockSpec((NW, 8, NPAD), lambda: (0, 0, 0))],
        out_specs=pl.BlockSpec((8, NPAD), lambda: (0, 0)),
        out_shape=jax.ShapeDtypeStruct((8, NPAD), _f32),
    )(pos8, cpart)


def kernel(h, pos, edge_index, t_emb, W_e1, b_e1, W_e2, b_e2, W_n1, b_n1,
           W_n2, b_n2, W_c1, b_c1, W_c2, gamma, beta):
    row = edge_index[0].astype(jnp.int32)
    col = edge_index[1].astype(jnp.int32)
    pad_idx = jnp.full((EPAD - E,), N, jnp.int32)
    row3 = jnp.concatenate([row, pad_idx]).reshape(NW, NCHUNK, CH)
    col3 = jnp.concatenate([col, pad_idx]).reshape(NW, NCHUNK, CH)

    hp = jnp.pad(h, ((0, NPAD - N), (0, 0)))
    tp = jnp.pad(t_emb, ((0, NPAD - N), (0, 0)))
    posTp = jnp.pad(pos.T, ((0, 0), (0, NPAD - N)))
    px, py, pz = posTp[0], posTp[1], posTp[2]

    WaT = W_e1[:, :HID].T
    WbT = W_e1[:, HID:2 * HID].T
    wd = W_e1[:, 2 * HID].reshape(1, HID)
    WtT = W_e1[:, 2 * HID + 1:].T

    A, B = _prep_call(hp, tp, WaT, WbT, WtT, b_e1.reshape(1, HID))
    GA, GB, dx, dy, dz = _gather_call(A, B, px, py, pz, row3, col3)
    mij, cwx, cwy, cwz = _edge_call(
        GA, GB, dx.reshape(EPAD, 1), dy.reshape(EPAD, 1), dz.reshape(EPAD, 1),
        W_e2.T, b_e2.reshape(1, HID), W_c1.T, b_c1.reshape(1, HID),
        W_c2.T, wd)
    mpart = _msg_scatter_call(row3, mij, jnp.zeros((NPAD, HID), _f32))
    cpart = _coord_scatter_call(
        row3, cwx.reshape(EPAD), cwy.reshape(EPAD), cwz.reshape(EPAD))

    h_final = _node_call(
        h, t_emb, mpart,
        W_n1[:, :HID].T, W_n1[:, HID:2 * HID].T, W_n1[:, 2 * HID:].T,
        b_n1.reshape(1, HID), W_n2.T, b_n2.reshape(1, HID),
        gamma.reshape(1, HID), beta.reshape(1, HID))

    pos8 = jnp.pad(posTp, ((0, 8 - 3), (0, 0)))
    out8 = _pos_call(pos8, cpart)
    pos_out = out8[:3, :N].T
    return h_final, pos_out


# speedup vs baseline: 1.0674x; 1.0674x over previous
import functools

import jax
import jax.numpy as jnp
from jax import lax
from jax.experimental import pallas as pl
from jax.experimental.pallas import tpu as pltpu
from jax.experimental.pallas import tpu_sc as plsc

N = 10000
E = 320000
HID = 128
NPAD = 10240
EPAD = 327680
CW = 16
NC = 2
NS = 16
NW = NC * NS
EPT = EPAD // NW
CH = 64
NCHUNK = EPT // CH
RPT = NPAD // NS

_f32 = jnp.float32


def _silu(x):
    return x / (1.0 + jnp.exp(-x))


def _prep_body(h_ref, t_ref, WaT, WbT, WtT, be1, A_ref, B_ref):
    h = h_ref[...]
    A_ref[...] = (jnp.dot(h, WaT[...], preferred_element_type=_f32)
                  + jnp.dot(t_ref[...], WtT[...], preferred_element_type=_f32)
                  + be1[...])
    B_ref[...] = jnp.dot(h, WbT[...], preferred_element_type=_f32)


def _prep_call(hp, tp, WaT, WbT, WtT, be1):
    blk = 2048
    grid = NPAD // blk
    w_spec = pl.BlockSpec((HID, HID), lambda i: (0, 0))
    b_spec = pl.BlockSpec((1, HID), lambda i: (0, 0))
    r_spec = pl.BlockSpec((blk, HID), lambda i: (i, 0))
    return pl.pallas_call(
        _prep_body,
        grid=(grid,),
        in_specs=[r_spec, r_spec, w_spec, w_spec, w_spec, b_spec],
        out_specs=[r_spec, r_spec],
        out_shape=[jax.ShapeDtypeStruct((NPAD, HID), _f32),
                   jax.ShapeDtypeStruct((NPAD, HID), _f32)],
    )(hp, tp, WaT, WbT, WtT, be1)


GCH = 128
GNCH = EPT // GCH


def _gather_sc(A, B, px, py, pz, row3, col3,
               GA, GB, dxo, dyo, dzo,
               pxt, pyt, pzt, rowf, colf,
               bufA0, bufA1, bufB0, bufB1,
               dxb0, dxb1, dyb0, dyb1, dzb0, dzb1,
               semA0, semA1, semB0, semB1, semO0, semO1):
    cid = lax.axis_index("c")
    sid = lax.axis_index("s")
    wid = cid * NS + sid
    tbase = pl.multiple_of(wid * EPT, GCH)
    pltpu.sync_copy(px, pxt)
    pltpu.sync_copy(py, pyt)
    pltpu.sync_copy(pz, pzt)
    pltpu.sync_copy(row3.at[wid], rowf)
    pltpu.sync_copy(col3.at[wid], colf)

    bufA = (bufA0, bufA1)
    bufB = (bufB0, bufB1)
    dxb = (dxb0, dxb1)
    dyb = (dyb0, dyb1)
    dzb = (dzb0, dzb1)
    semA = (semA0, semA1)
    semB = (semB0, semB1)
    semO = (semO0, semO1)

    def out_descs(k, b):
        base = tbase + k * GCH
        return (
            pltpu.make_async_copy(bufA[b], GA.at[pl.ds(base, GCH)], semO[b]),
            pltpu.make_async_copy(bufB[b], GB.at[pl.ds(base, GCH)], semO[b]),
            pltpu.make_async_copy(dxb[b], dxo.at[pl.ds(base, GCH)], semO[b]),
            pltpu.make_async_copy(dyb[b], dyo.at[pl.ds(base, GCH)], semO[b]),
            pltpu.make_async_copy(dzb[b], dzo.at[pl.ds(base, GCH)], semO[b]),
        )

    def issue_in(k, b):
        pltpu.make_async_copy(A.at[rowf.at[k]], bufA[b], semA[b]).start()
        pltpu.make_async_copy(B.at[colf.at[k]], bufB[b], semB[b]).start()

    def wait_in(k, b):
        pltpu.make_async_copy(A.at[rowf.at[k]], bufA[b], semA[b]).wait()
        pltpu.make_async_copy(B.at[colf.at[k]], bufB[b], semB[b]).wait()

    def drain_out(k, b):
        for d in out_descs(k, b):
            d.wait()

    issue_in(0, 0)

    def body(i, carry):
        for b in (0, 1):
            k = 2 * i + b
            wait_in(k, b)
            if b == 0:
                @pl.when(i >= 1)
                def _():
                    drain_out(k - 1, 1)
                issue_in(k + 1, 1)
            else:
                drain_out(k - 1, 0)

                @pl.when(i < GNCH // 2 - 1)
                def _():
                    issue_in(k + 1, 0)
            for j in range(GCH // 16):
                r = rowf[k, pl.ds(j * 16, 16)]
                c = colf[k, pl.ds(j * 16, 16)]
                o = pl.ds(j * 16, 16)
                dxb[b][o] = (plsc.load_gather(pxt, [r])
                             - plsc.load_gather(pxt, [c]))
                dyb[b][o] = (plsc.load_gather(pyt, [r])
                             - plsc.load_gather(pyt, [c]))
                dzb[b][o] = (plsc.load_gather(pzt, [r])
                             - plsc.load_gather(pzt, [c]))
            for d in out_descs(k, b):
                d.start()
        return carry

    lax.fori_loop(0, GNCH // 2, body, 0)
    drain_out(GNCH - 1, 1)


def _gather_call(A, B, px, py, pz, row3, col3):
    mesh = plsc.VectorSubcoreMesh(core_axis_name="c", subcore_axis_name="s",
                                  num_cores=NC, num_subcores=NS)
    f = functools.partial(
        pl.kernel,
        out_type=[jax.ShapeDtypeStruct((EPAD, HID), _f32),
                  jax.ShapeDtypeStruct((EPAD, HID), _f32),
                  jax.ShapeDtypeStruct((EPAD,), _f32),
                  jax.ShapeDtypeStruct((EPAD,), _f32),
                  jax.ShapeDtypeStruct((EPAD,), _f32)],
        mesh=mesh,
        scratch_types=[
            pltpu.VMEM((NPAD,), _f32),
            pltpu.VMEM((NPAD,), _f32),
            pltpu.VMEM((NPAD,), _f32),
            pltpu.VMEM((GNCH, GCH), jnp.int32),
            pltpu.VMEM((GNCH, GCH), jnp.int32),
            pltpu.VMEM((GCH, HID), _f32),
            pltpu.VMEM((GCH, HID), _f32),
            pltpu.VMEM((GCH, HID), _f32),
            pltpu.VMEM((GCH, HID), _f32),
            pltpu.VMEM((GCH,), _f32),
            pltpu.VMEM((GCH,), _f32),
            pltpu.VMEM((GCH,), _f32),
            pltpu.VMEM((GCH,), _f32),
            pltpu.VMEM((GCH,), _f32),
            pltpu.VMEM((GCH,), _f32),
            pltpu.SemaphoreType.DMA,
            pltpu.SemaphoreType.DMA,
            pltpu.SemaphoreType.DMA,
            pltpu.SemaphoreType.DMA,
            pltpu.SemaphoreType.DMA,
            pltpu.SemaphoreType.DMA,
        ],
        compiler_params=pltpu.CompilerParams(needs_layout_passes=False),
    )(_gather_sc)
    return f(A, B, px, py, pz, row3, col3)


def _edge_body(GA, GB, dx, dy, dz, We2T, be2, Wc1T, bc1, Wc2T, wd,
               mij_o, cwx_o, cwy_o, cwz_o):
    dxv = dx[...]
    dyv = dy[...]
    dzv = dz[...]
    d2 = dxv * dxv + dyv * dyv + dzv * dzv + 1e-8
    dist = jnp.sqrt(d2)
    pre = GA[...] + GB[...] + dist * wd[...]
    m = _silu(pre)
    mij = _silu(jnp.dot(m, We2T[...], preferred_element_type=_f32) + be2[...])
    cwv = _silu(jnp.dot(mij, Wc1T[...], preferred_element_type=_f32)
                + bc1[...])
    cw = jnp.dot(cwv, Wc2T[...], preferred_element_type=_f32)
    s = cw / (dist + 1e-8)
    mij_o[...] = mij
    cwx_o[...] = s * dxv
    cwy_o[...] = s * dyv
    cwz_o[...] = s * dzv


def _edge_call(GA, GB, dx, dy, dz, We2T, be2, Wc1T, bc1, Wc2T, wd):
    blk = 1024
    grid = EPAD // blk
    g_spec = pl.BlockSpec((blk, HID), lambda i: (i, 0))
    c_spec = pl.BlockSpec((blk, 1), lambda i: (i, 0))
    w_spec = pl.BlockSpec((HID, HID), lambda i: (0, 0))
    b_spec = pl.BlockSpec((1, HID), lambda i: (0, 0))
    w2_spec = pl.BlockSpec((HID, 1), lambda i: (0, 0))
    return pl.pallas_call(
        _edge_body,
        grid=(grid,),
        in_specs=[g_spec, g_spec, c_spec, c_spec, c_spec, w_spec, b_spec,
                  w_spec, b_spec, w2_spec, b_spec],
        out_specs=[g_spec, c_spec, c_spec, c_spec],
        out_shape=[jax.ShapeDtypeStruct((EPAD, HID), _f32),
                   jax.ShapeDtypeStruct((EPAD, 1), _f32),
                   jax.ShapeDtypeStruct((EPAD, 1), _f32),
                   jax.ShapeDtypeStruct((EPAD, 1), _f32)],
    )(GA, GB, dx, dy, dz, We2T, be2, Wc1T, bc1, Wc2T, wd)


def _msg_scatter_sc(row3, mij, z2d, mpart, macc, rowf, mbuf):
    cid = lax.axis_index("c")
    sid = lax.axis_index("s")
    wid = cid * NS + sid
    pltpu.sync_copy(z2d.at[pl.ds(sid * RPT, RPT)],
                    macc.at[pl.ds(sid * RPT, RPT)])
    pltpu.sync_copy(row3.at[wid], rowf)
    plsc.subcore_barrier()

    def chunk(k, carry):
        base = pl.multiple_of(wid * EPT, CH) + k * CH
        pltpu.sync_copy(mij.at[pl.ds(base, CH)], mbuf)
        pltpu.sync_copy(mbuf, macc.at[rowf.at[k]], add=True)
        return carry

    lax.fori_loop(0, NCHUNK, chunk, 0)
    plsc.subcore_barrier()
    pltpu.sync_copy(macc.at[pl.ds(sid * RPT, RPT)],
                    mpart.at[cid, pl.ds(sid * RPT, RPT)])


def _msg_scatter_call(row3, mij, z2d):
    mesh = plsc.VectorSubcoreMesh(core_axis_name="c", subcore_axis_name="s",
                                  num_cores=NC, num_subcores=NS)
    f = functools.partial(
        pl.kernel,
        out_type=[jax.ShapeDtypeStruct((NC, NPAD, HID), _f32)],
        mesh=mesh,
        scratch_types=[
            pltpu.MemorySpace.VMEM_SHARED((NPAD, HID), _f32),
            pltpu.VMEM((NCHUNK, CH), jnp.int32),
            pltpu.VMEM((CH, HID), _f32),
        ],
        compiler_params=pltpu.CompilerParams(needs_layout_passes=False),
    )(_msg_scatter_sc)
    [mpart] = f(row3, mij, z2d)
    return mpart


def _coord_scatter_sc(row3, cwx, cwy, cwz, cpart,
                      acc3, rowf, cxb, cyb, czb):
    cid = lax.axis_index("c")
    sid = lax.axis_index("s")
    wid = cid * NS + sid

    def zinit(i, c2):
        o = pl.ds(i * 16, 16)
        z = jnp.zeros((16,), _f32)
        acc3[0, o] = z
        acc3[1, o] = z
        acc3[2, o] = z
        return c2

    lax.fori_loop(0, NPAD // 16, zinit, 0)
    tbase = pl.multiple_of(wid * EPT, CH)
    pltpu.sync_copy(row3.at[wid], rowf)
    pltpu.sync_copy(cwx.at[pl.ds(tbase, EPT)], cxb)
    pltpu.sync_copy(cwy.at[pl.ds(tbase, EPT)], cyb)
    pltpu.sync_copy(cwz.at[pl.ds(tbase, EPT)], czb)
    d0 = jnp.zeros((16,), jnp.int32)
    d1 = jnp.ones((16,), jnp.int32)
    d2 = jnp.full((16,), 2, jnp.int32)

    def chunk(k, carry):
        def vec(i, c2):
            o = pl.ds(k * CH + i * 16, 16)
            r = rowf[k, pl.ds(i * 16, 16)]
            plsc.addupdate_scatter(acc3, [d0, r], cxb[o])
            plsc.addupdate_scatter(acc3, [d1, r], cyb[o])
            plsc.addupdate_scatter(acc3, [d2, r], czb[o])
            return c2

        return lax.fori_loop(0, CH // 16, vec, carry)

    lax.fori_loop(0, NCHUNK, chunk, 0)
    pltpu.sync_copy(acc3, cpart.at[wid, pl.ds(0, 3)])


def _coord_scatter_call(row3, cwx, cwy, cwz):
    mesh = plsc.VectorSubcoreMesh(core_axis_name="c", subcore_axis_name="s",
                                  num_cores=NC, num_subcores=NS)
    f = functools.partial(
        pl.kernel,
        out_type=[jax.ShapeDtypeStruct((NW, 8, NPAD), _f32)],
        mesh=mesh,
        scratch_types=[
            pltpu.VMEM((3, NPAD), _f32),
            pltpu.VMEM((NCHUNK, CH), jnp.int32),
            pltpu.VMEM((EPT,), _f32),
            pltpu.VMEM((EPT,), _f32),
            pltpu.VMEM((EPT,), _f32),
        ],
        compiler_params=pltpu.CompilerParams(needs_layout_passes=False),
    )(_coord_scatter_sc)
    [cpart] = f(row3, cwx, cwy, cwz)
    return cpart


def _node_body(h_ref, t_ref, p0, p1, WhT, WmT, Wt2T, bn1, Wn2T, bn2, gam, bet,
               out):
    msg = p0[0] + p1[0]
    hv = h_ref[...]
    hm = _silu(jnp.dot(hv, WhT[...], preferred_element_type=_f32)
               + jnp.dot(msg, WmT[...], preferred_element_type=_f32)
               + jnp.dot(t_ref[...], Wt2T[...], preferred_element_type=_f32)
               + bn1[...])
    y = hv + jnp.dot(hm, Wn2T[...], preferred_element_type=_f32) + bn2[...]
    mu = jnp.mean(y, axis=1, keepdims=True)
    var = jnp.mean((y - mu) ** 2, axis=1, keepdims=True)
    out[...] = (y - mu) * lax.rsqrt(var + 1e-5) * gam[...] + bet[...]


def _node_call(h, t_emb, mpart, WhT, WmT, Wt2T, bn1, Wn2T, bn2, gam, bet):
    blk = 2000
    grid = N // blk
    r_spec = pl.BlockSpec((blk, HID), lambda i: (i, 0))
    p0_spec = pl.BlockSpec((1, blk, HID), lambda i: (0, i, 0))
    p1_spec = pl.BlockSpec((1, blk, HID), lambda i: (1, i, 0))
    w_spec = pl.BlockSpec((HID, HID), lambda i: (0, 0))
    b_spec = pl.BlockSpec((1, HID), lambda i: (0, 0))
    return pl.pallas_call(
        _node_body,
        grid=(grid,),
        in_specs=[r_spec, r_spec, p0_spec, p1_spec, w_spec, w_spec, w_spec,
                  b_spec, w_spec, b_spec, b_spec, b_spec],
        out_specs=r_spec,
        out_shape=jax.ShapeDtypeStruct((N, HID), _f32),
    )(h, t_emb, mpart, mpart, WhT, WmT, Wt2T, bn1, Wn2T, bn2, gam, bet)


def _pos_body(pos8, cp, out):
    out[...] = pos8[...] + jnp.sum(cp[...], axis=0)


def _pos_call(pos8, cpart):
    return pl.pallas_call(
        _pos_body,
        in_specs=[pl.BlockSpec((8, NPAD), lambda: (0, 0)),
                  pl.BlockSpec((NW, 8, NPAD), lambda: (0, 0, 0))],
        out_specs=pl.BlockSpec((8, NPAD), lambda: (0, 0)),
        out_shape=jax.ShapeDtypeStruct((8, NPAD), _f32),
    )(pos8, cpart)


def kernel(h, pos, edge_index, t_emb, W_e1, b_e1, W_e2, b_e2, W_n1, b_n1,
           W_n2, b_n2, W_c1, b_c1, W_c2, gamma, beta):
    row = edge_index[0].astype(jnp.int32)
    col = edge_index[1].astype(jnp.int32)
    pad_idx = jnp.full((EPAD - E,), N, jnp.int32)
    rowp = jnp.concatenate([row, pad_idx])
    colp = jnp.concatenate([col, pad_idx])
    row3 = rowp.reshape(NW, NCHUNK, CH)
    row3g = rowp.reshape(NW, GNCH, GCH)
    col3g = colp.reshape(NW, GNCH, GCH)

    hp = jnp.pad(h, ((0, NPAD - N), (0, 0)))
    tp = jnp.pad(t_emb, ((0, NPAD - N), (0, 0)))
    posTp = jnp.pad(pos.T, ((0, 0), (0, NPAD - N)))
    px, py, pz = posTp[0], posTp[1], posTp[2]

    WaT = W_e1[:, :HID].T
    WbT = W_e1[:, HID:2 * HID].T
    wd = W_e1[:, 2 * HID].reshape(1, HID)
    WtT = W_e1[:, 2 * HID + 1:].T

    A, B = _prep_call(hp, tp, WaT, WbT, WtT, b_e1.reshape(1, HID))
    GA, GB, dx, dy, dz = _gather_call(A, B, px, py, pz, row3g, col3g)
    mij, cwx, cwy, cwz = _edge_call(
        GA, GB, dx.reshape(EPAD, 1), dy.reshape(EPAD, 1), dz.reshape(EPAD, 1),
        W_e2.T, b_e2.reshape(1, HID), W_c1.T, b_c1.reshape(1, HID),
        W_c2.T, wd)
    mpart = _msg_scatter_call(row3, mij, jnp.zeros((NPAD, HID), _f32))
    cpart = _coord_scatter_call(
        row3, cwx.reshape(EPAD), cwy.reshape(EPAD), cwz.reshape(EPAD))

    h_final = _node_call(
        h, t_emb, mpart,
        W_n1[:, :HID].T, W_n1[:, HID:2 * HID].T, W_n1[:, 2 * HID:].T,
        b_n1.reshape(1, HID), W_n2.T, b_n2.reshape(1, HID),
        gamma.reshape(1, HID), beta.reshape(1, HID))

    pos8 = jnp.pad(posTp, ((0, 8 - 3), (0, 0)))
    out8 = _pos_call(pos8, cpart)
    pos_out = out8[:3, :N].T
    return h_final, pos_out


# speedup vs baseline: 1.1185x; 1.0479x over previous
import functools

import jax
import jax.numpy as jnp
from jax import lax
from jax.experimental import pallas as pl
from jax.experimental.pallas import tpu as pltpu
from jax.experimental.pallas import tpu_sc as plsc

N = 10000
E = 320000
HID = 128
NPAD = 10240
EPAD = 327680
CW = 16
NC = 2
NS = 16
NW = NC * NS
EPT = EPAD // NW
CH = 64
NCHUNK = EPT // CH
RPT = NPAD // NS

_f32 = jnp.float32


def _silu(x):
    return x / (1.0 + jnp.exp(-x))


def _prep_body(h_ref, t_ref, WaT, WbT, WtT, be1, A_ref, B_ref):
    h = h_ref[...]
    A_ref[...] = (jnp.dot(h, WaT[...], preferred_element_type=_f32)
                  + jnp.dot(t_ref[...], WtT[...], preferred_element_type=_f32)
                  + be1[...])
    B_ref[...] = jnp.dot(h, WbT[...], preferred_element_type=_f32)


def _prep_call(hp, tp, WaT, WbT, WtT, be1):
    blk = 2048
    grid = NPAD // blk
    w_spec = pl.BlockSpec((HID, HID), lambda i: (0, 0))
    b_spec = pl.BlockSpec((1, HID), lambda i: (0, 0))
    r_spec = pl.BlockSpec((blk, HID), lambda i: (i, 0))
    return pl.pallas_call(
        _prep_body,
        grid=(grid,),
        in_specs=[r_spec, r_spec, w_spec, w_spec, w_spec, b_spec],
        out_specs=[r_spec, r_spec],
        out_shape=[jax.ShapeDtypeStruct((NPAD, HID), _f32),
                   jax.ShapeDtypeStruct((NPAD, HID), _f32)],
    )(hp, tp, WaT, WbT, WtT, be1)


GCH = 128
GNCH = EPT // GCH


def _gather_sc(A, B, px, py, pz, row3, col3,
               G, dxo, dyo, dzo,
               pxt, pyt, pzt, rowf, colf,
               bufA0, bufA1, bufB0, bufB1,
               dxb0, dxb1, dyb0, dyb1, dzb0, dzb1,
               semA0, semA1, semB0, semB1, semO0, semO1):
    cid = lax.axis_index("c")
    sid = lax.axis_index("s")
    wid = cid * NS + sid
    tbase = pl.multiple_of(wid * EPT, GCH)
    pltpu.sync_copy(px, pxt)
    pltpu.sync_copy(py, pyt)
    pltpu.sync_copy(pz, pzt)
    pltpu.sync_copy(row3.at[wid], rowf)
    pltpu.sync_copy(col3.at[wid], colf)

    bufA = (bufA0, bufA1)
    bufB = (bufB0, bufB1)
    dxb = (dxb0, dxb1)
    dyb = (dyb0, dyb1)
    dzb = (dzb0, dzb1)
    semA = (semA0, semA1)
    semB = (semB0, semB1)
    semO = (semO0, semO1)

    def out_descs(k, b):
        base = tbase + k * GCH
        return (
            pltpu.make_async_copy(bufA[b], G.at[pl.ds(base, GCH)], semO[b]),
            pltpu.make_async_copy(dxb[b], dxo.at[pl.ds(base, GCH)], semO[b]),
            pltpu.make_async_copy(dyb[b], dyo.at[pl.ds(base, GCH)], semO[b]),
            pltpu.make_async_copy(dzb[b], dzo.at[pl.ds(base, GCH)], semO[b]),
        )

    def issue_in(k, b):
        pltpu.make_async_copy(A.at[rowf.at[k]], bufA[b], semA[b]).start()
        pltpu.make_async_copy(B.at[colf.at[k]], bufB[b], semB[b]).start()

    def wait_in(k, b):
        pltpu.make_async_copy(A.at[rowf.at[k]], bufA[b], semA[b]).wait()
        pltpu.make_async_copy(B.at[colf.at[k]], bufB[b], semB[b]).wait()

    def drain_out(k, b):
        for d in out_descs(k, b):
            d.wait()

    issue_in(0, 0)

    def body(i, carry):
        for b in (0, 1):
            k = 2 * i + b
            wait_in(k, b)
            if b == 0:
                @pl.when(i >= 1)
                def _():
                    drain_out(k - 1, 1)
                issue_in(k + 1, 1)
            else:
                drain_out(k - 1, 0)

                @pl.when(i < GNCH // 2 - 1)
                def _():
                    issue_in(k + 1, 0)
            for j in range(GCH // 16):
                r = rowf[k, pl.ds(j * 16, 16)]
                c = colf[k, pl.ds(j * 16, 16)]
                o = pl.ds(j * 16, 16)
                dxb[b][o] = (plsc.load_gather(pxt, [r])
                             - plsc.load_gather(pxt, [c]))
                dyb[b][o] = (plsc.load_gather(pyt, [r])
                             - plsc.load_gather(pyt, [c]))
                dzb[b][o] = (plsc.load_gather(pzt, [r])
                             - plsc.load_gather(pzt, [c]))

            def addrow(i2, c2):
                for j2 in range(HID // 16):
                    s2 = pl.ds(j2 * 16, 16)
                    bufA[b][i2, s2] = bufA[b][i2, s2] + bufB[b][i2, s2]
                return c2

            lax.fori_loop(0, GCH, addrow, 0)
            for d in out_descs(k, b):
                d.start()
        return carry

    lax.fori_loop(0, GNCH // 2, body, 0)
    drain_out(GNCH - 1, 1)


def _gather_call(A, B, px, py, pz, row3, col3):
    mesh = plsc.VectorSubcoreMesh(core_axis_name="c", subcore_axis_name="s",
                                  num_cores=NC, num_subcores=NS)
    f = functools.partial(
        pl.kernel,
        out_type=[jax.ShapeDtypeStruct((EPAD, HID), _f32),
                  jax.ShapeDtypeStruct((EPAD,), _f32),
                  jax.ShapeDtypeStruct((EPAD,), _f32),
                  jax.ShapeDtypeStruct((EPAD,), _f32)],
        mesh=mesh,
        scratch_types=[
            pltpu.VMEM((NPAD,), _f32),
            pltpu.VMEM((NPAD,), _f32),
            pltpu.VMEM((NPAD,), _f32),
            pltpu.VMEM((GNCH, GCH), jnp.int32),
            pltpu.VMEM((GNCH, GCH), jnp.int32),
            pltpu.VMEM((GCH, HID), _f32),
            pltpu.VMEM((GCH, HID), _f32),
            pltpu.VMEM((GCH, HID), _f32),
            pltpu.VMEM((GCH, HID), _f32),
            pltpu.VMEM((GCH,), _f32),
            pltpu.VMEM((GCH,), _f32),
            pltpu.VMEM((GCH,), _f32),
            pltpu.VMEM((GCH,), _f32),
            pltpu.VMEM((GCH,), _f32),
            pltpu.VMEM((GCH,), _f32),
            pltpu.SemaphoreType.DMA,
            pltpu.SemaphoreType.DMA,
            pltpu.SemaphoreType.DMA,
            pltpu.SemaphoreType.DMA,
            pltpu.SemaphoreType.DMA,
            pltpu.SemaphoreType.DMA,
        ],
        compiler_params=pltpu.CompilerParams(needs_layout_passes=False),
    )(_gather_sc)
    return f(A, B, px, py, pz, row3, col3)


def _edge_body(G, dx, dy, dz, We2T, be2, Wc1T, bc1, Wc2T, wd,
               mij_o, cwx_o, cwy_o, cwz_o):
    dxv = dx[...]
    dyv = dy[...]
    dzv = dz[...]
    d2 = dxv * dxv + dyv * dyv + dzv * dzv + 1e-8
    dist = jnp.sqrt(d2)
    pre = G[...] + dist * wd[...]
    m = _silu(pre)
    mij = _silu(jnp.dot(m, We2T[...], preferred_element_type=_f32) + be2[...])
    cwv = _silu(jnp.dot(mij, Wc1T[...], preferred_element_type=_f32)
                + bc1[...])
    cw = jnp.dot(cwv, Wc2T[...], preferred_element_type=_f32)
    s = cw / (dist + 1e-8)
    mij_o[...] = mij
    cwx_o[...] = s * dxv
    cwy_o[...] = s * dyv
    cwz_o[...] = s * dzv


def _edge_call(G, dx, dy, dz, We2T, be2, Wc1T, bc1, Wc2T, wd):
    blk = 1024
    grid = EPAD // blk
    g_spec = pl.BlockSpec((blk, HID), lambda i: (i, 0))
    c_spec = pl.BlockSpec((blk, 1), lambda i: (i, 0))
    w_spec = pl.BlockSpec((HID, HID), lambda i: (0, 0))
    b_spec = pl.BlockSpec((1, HID), lambda i: (0, 0))
    w2_spec = pl.BlockSpec((HID, 1), lambda i: (0, 0))
    return pl.pallas_call(
        _edge_body,
        grid=(grid,),
        in_specs=[g_spec, c_spec, c_spec, c_spec, w_spec, b_spec,
                  w_spec, b_spec, w2_spec, b_spec],
        out_specs=[g_spec, c_spec, c_spec, c_spec],
        out_shape=[jax.ShapeDtypeStruct((EPAD, HID), _f32),
                   jax.ShapeDtypeStruct((EPAD, 1), _f32),
                   jax.ShapeDtypeStruct((EPAD, 1), _f32),
                   jax.ShapeDtypeStruct((EPAD, 1), _f32)],
    )(G, dx, dy, dz, We2T, be2, Wc1T, bc1, Wc2T, wd)


def _msg_scatter_sc(row3, mij, z2d, mpart, macc, rowf, mbuf):
    cid = lax.axis_index("c")
    sid = lax.axis_index("s")
    wid = cid * NS + sid
    pltpu.sync_copy(z2d.at[pl.ds(sid * RPT, RPT)],
                    macc.at[pl.ds(sid * RPT, RPT)])
    pltpu.sync_copy(row3.at[wid], rowf)
    plsc.subcore_barrier()

    def chunk(k, carry):
        base = pl.multiple_of(wid * EPT, CH) + k * CH
        pltpu.sync_copy(mij.at[pl.ds(base, CH)], mbuf)
        pltpu.sync_copy(mbuf, macc.at[rowf.at[k]], add=True)
        return carry

    lax.fori_loop(0, NCHUNK, chunk, 0)
    plsc.subcore_barrier()
    pltpu.sync_copy(macc.at[pl.ds(sid * RPT, RPT)],
                    mpart.at[cid, pl.ds(sid * RPT, RPT)])


def _msg_scatter_call(row3, mij, z2d):
    mesh = plsc.VectorSubcoreMesh(core_axis_name="c", subcore_axis_name="s",
                                  num_cores=NC, num_subcores=NS)
    f = functools.partial(
        pl.kernel,
        out_type=[jax.ShapeDtypeStruct((NC, NPAD, HID), _f32)],
        mesh=mesh,
        scratch_types=[
            pltpu.MemorySpace.VMEM_SHARED((NPAD, HID), _f32),
            pltpu.VMEM((NCHUNK, CH), jnp.int32),
            pltpu.VMEM((CH, HID), _f32),
        ],
        compiler_params=pltpu.CompilerParams(needs_layout_passes=False),
    )(_msg_scatter_sc)
    [mpart] = f(row3, mij, z2d)
    return mpart


def _coord_scatter_sc(row3, cwx, cwy, cwz, cpart,
                      acc3, rowf, cxb, cyb, czb):
    cid = lax.axis_index("c")
    sid = lax.axis_index("s")
    wid = cid * NS + sid

    def zinit(i, c2):
        o = pl.ds(i * 16, 16)
        z = jnp.zeros((16,), _f32)
        acc3[0, o] = z
        acc3[1, o] = z
        acc3[2, o] = z
        return c2

    lax.fori_loop(0, NPAD // 16, zinit, 0)
    tbase = pl.multiple_of(wid * EPT, CH)
    pltpu.sync_copy(row3.at[wid], rowf)
    pltpu.sync_copy(cwx.at[pl.ds(tbase, EPT)], cxb)
    pltpu.sync_copy(cwy.at[pl.ds(tbase, EPT)], cyb)
    pltpu.sync_copy(cwz.at[pl.ds(tbase, EPT)], czb)
    d0 = jnp.zeros((16,), jnp.int32)
    d1 = jnp.ones((16,), jnp.int32)
    d2 = jnp.full((16,), 2, jnp.int32)

    def chunk(k, carry):
        def vec(i, c2):
            o = pl.ds(k * CH + i * 16, 16)
            r = rowf[k, pl.ds(i * 16, 16)]
            plsc.addupdate_scatter(acc3, [d0, r], cxb[o])
            plsc.addupdate_scatter(acc3, [d1, r], cyb[o])
            plsc.addupdate_scatter(acc3, [d2, r], czb[o])
            return c2

        return lax.fori_loop(0, CH // 16, vec, carry)

    lax.fori_loop(0, NCHUNK, chunk, 0)
    pltpu.sync_copy(acc3, cpart.at[wid, pl.ds(0, 3)])


def _coord_scatter_call(row3, cwx, cwy, cwz):
    mesh = plsc.VectorSubcoreMesh(core_axis_name="c", subcore_axis_name="s",
                                  num_cores=NC, num_subcores=NS)
    f = functools.partial(
        pl.kernel,
        out_type=[jax.ShapeDtypeStruct((NW, 8, NPAD), _f32)],
        mesh=mesh,
        scratch_types=[
            pltpu.VMEM((3, NPAD), _f32),
            pltpu.VMEM((NCHUNK, CH), jnp.int32),
            pltpu.VMEM((EPT,), _f32),
            pltpu.VMEM((EPT,), _f32),
            pltpu.VMEM((EPT,), _f32),
        ],
        compiler_params=pltpu.CompilerParams(needs_layout_passes=False),
    )(_coord_scatter_sc)
    [cpart] = f(row3, cwx, cwy, cwz)
    return cpart


def _node_body(h_ref, t_ref, p0, p1, WhT, WmT, Wt2T, bn1, Wn2T, bn2, gam, bet,
               out):
    msg = p0[0] + p1[0]
    hv = h_ref[...]
    hm = _silu(jnp.dot(hv, WhT[...], preferred_element_type=_f32)
               + jnp.dot(msg, WmT[...], preferred_element_type=_f32)
               + jnp.dot(t_ref[...], Wt2T[...], preferred_element_type=_f32)
               + bn1[...])
    y = hv + jnp.dot(hm, Wn2T[...], preferred_element_type=_f32) + bn2[...]
    mu = jnp.mean(y, axis=1, keepdims=True)
    var = jnp.mean((y - mu) ** 2, axis=1, keepdims=True)
    out[...] = (y - mu) * lax.rsqrt(var + 1e-5) * gam[...] + bet[...]


def _node_call(h, t_emb, mpart, WhT, WmT, Wt2T, bn1, Wn2T, bn2, gam, bet):
    blk = 2000
    grid = N // blk
    r_spec = pl.BlockSpec((blk, HID), lambda i: (i, 0))
    p0_spec = pl.BlockSpec((1, blk, HID), lambda i: (0, i, 0))
    p1_spec = pl.BlockSpec((1, blk, HID), lambda i: (1, i, 0))
    w_spec = pl.BlockSpec((HID, HID), lambda i: (0, 0))
    b_spec = pl.BlockSpec((1, HID), lambda i: (0, 0))
    return pl.pallas_call(
        _node_body,
        grid=(grid,),
        in_specs=[r_spec, r_spec, p0_spec, p1_spec, w_spec, w_spec, w_spec,
                  b_spec, w_spec, b_spec, b_spec, b_spec],
        out_specs=r_spec,
        out_shape=jax.ShapeDtypeStruct((N, HID), _f32),
    )(h, t_emb, mpart, mpart, WhT, WmT, Wt2T, bn1, Wn2T, bn2, gam, bet)


def _pos_body(pos8, cp, out):
    out[...] = pos8[...] + jnp.sum(cp[...], axis=0)


def _pos_call(pos8, cpart):
    return pl.pallas_call(
        _pos_body,
        in_specs=[pl.BlockSpec((8, NPAD), lambda: (0, 0)),
                  pl.BlockSpec((NW, 8, NPAD), lambda: (0, 0, 0))],
        out_specs=pl.BlockSpec((8, NPAD), lambda: (0, 0)),
        out_shape=jax.ShapeDtypeStruct((8, NPAD), _f32),
    )(pos8, cpart)


def kernel(h, pos, edge_index, t_emb, W_e1, b_e1, W_e2, b_e2, W_n1, b_n1,
           W_n2, b_n2, W_c1, b_c1, W_c2, gamma, beta):
    row = edge_index[0].astype(jnp.int32)
    col = edge_index[1].astype(jnp.int32)
    pad_idx = jnp.full((EPAD - E,), N, jnp.int32)
    rowp = jnp.concatenate([row, pad_idx])
    colp = jnp.concatenate([col, pad_idx])
    row3 = rowp.reshape(NW, NCHUNK, CH)
    row3g = rowp.reshape(NW, GNCH, GCH)
    col3g = colp.reshape(NW, GNCH, GCH)

    hp = jnp.pad(h, ((0, NPAD - N), (0, 0)))
    tp = jnp.pad(t_emb, ((0, NPAD - N), (0, 0)))
    posTp = jnp.pad(pos.T, ((0, 0), (0, NPAD - N)))
    px, py, pz = posTp[0], posTp[1], posTp[2]

    WaT = W_e1[:, :HID].T
    WbT = W_e1[:, HID:2 * HID].T
    wd = W_e1[:, 2 * HID].reshape(1, HID)
    WtT = W_e1[:, 2 * HID + 1:].T

    A, B = _prep_call(hp, tp, WaT, WbT, WtT, b_e1.reshape(1, HID))
    G, dx, dy, dz = _gather_call(A, B, px, py, pz, row3g, col3g)
    mij, cwx, cwy, cwz = _edge_call(
        G, dx.reshape(EPAD, 1), dy.reshape(EPAD, 1), dz.reshape(EPAD, 1),
        W_e2.T, b_e2.reshape(1, HID), W_c1.T, b_c1.reshape(1, HID),
        W_c2.T, wd)
    mpart = _msg_scatter_call(row3, mij, jnp.zeros((NPAD, HID), _f32))
    cpart = _coord_scatter_call(
        row3, cwx.reshape(EPAD), cwy.reshape(EPAD), cwz.reshape(EPAD))

    h_final = _node_call(
        h, t_emb, mpart,
        W_n1[:, :HID].T, W_n1[:, HID:2 * HID].T, W_n1[:, 2 * HID:].T,
        b_n1.reshape(1, HID), W_n2.T, b_n2.reshape(1, HID),
        gamma.reshape(1, HID), beta.reshape(1, HID))

    pos8 = jnp.pad(posTp, ((0, 8 - 3), (0, 0)))
    out8 = _pos_call(pos8, cpart)
    pos_out = out8[:3, :N].T
    return h_final, pos_out
